# baseline (speedup 1.0000x reference)
import jax
import jax.numpy as jnp
from jax.experimental import pallas as pl
from jax.experimental.pallas import tpu as pltpu

B, T, D, E = 4, 4096, 4096, 64
ROWS = 16384
BLK = 512
NBUF = 4


def _outer(x_hbm, sigt_ref, idx_hbm, probs_hbm, sgn_ref):
    sgn_ref[...] = jnp.sign(sigt_ref[...]).astype(jnp.bfloat16)

    def step(x_ref, idx_ref, probs_ref):
        xb = x_ref[...].astype(jnp.bfloat16)
        scores = jnp.dot(xb, sgn_ref[...],
                         preferred_element_type=jnp.float32)
        m = jnp.max(scores, axis=1, keepdims=True)
        col = jax.lax.broadcasted_iota(jnp.int32, scores.shape, 1)
        idx_ref[...] = jnp.min(jnp.where(scores == m, col, E), axis=1,
                               keepdims=True)
        e = jnp.exp(scores - m)
        probs_ref[...] = e / jnp.sum(e, axis=1, keepdims=True)

    pipeline = pltpu.emit_pipeline(
        step,
        grid=(ROWS // BLK,),
        in_specs=[
            pl.BlockSpec((BLK, D), lambda i: (i, 0),
                         pipeline_mode=pl.Buffered(buffer_count=NBUF)),
        ],
        out_specs=[
            pl.BlockSpec((BLK, 1), lambda i: (i, 0)),
            pl.BlockSpec((BLK, E), lambda i: (i, 0)),
        ],
    )
    pipeline(x_hbm, idx_hbm, probs_hbm)


def kernel(x, signatures):
    x2 = x.reshape(ROWS, D)
    sigt = signatures.T

    idx, probs = pl.pallas_call(
        _outer,
        in_specs=[
            pl.BlockSpec(memory_space=pl.ANY),
            pl.BlockSpec(memory_space=pltpu.VMEM),
        ],
        out_specs=[
            pl.BlockSpec(memory_space=pl.ANY),
            pl.BlockSpec(memory_space=pl.ANY),
        ],
        out_shape=[
            jax.ShapeDtypeStruct((ROWS, 1), jnp.int32),
            jax.ShapeDtypeStruct((ROWS, E), jnp.float32),
        ],
        scratch_shapes=[pltpu.VMEM((D, E), jnp.bfloat16)],
        compiler_params=pltpu.CompilerParams(
            vmem_limit_bytes=100 * 1024 * 1024,
        ),
    )(x2, sigt)

    return idx.reshape(B, T), probs.reshape(B, T, E)
